# baseline (speedup 1.0000x reference)
import jax
import jax.numpy as jnp
from jax.experimental import pallas as pl


def _readout_kernel(x_ref, oh_ref, w_ref, b_ref, out_ref):
    x = x_ref[...]
    T = out_ref.shape[0]
    for t in range(T):
        acc = jnp.dot(x, w_ref[t], preferred_element_type=jnp.float32)
        mask = oh_ref[:, t:t + 1]
        out_ref[t] = (acc + b_ref[t]) * mask


def kernel(output_latents, output_task_index, W, b):
    B, N, D = output_latents.shape
    T, _, E = W.shape
    M = B * N

    X = output_latents.reshape(M, D)
    idx = output_task_index.reshape(M)
    onehot = (idx[:, None] == jnp.arange(T, dtype=idx.dtype)[None, :]).astype(
        jnp.float32)

    TM = 512
    grid = (M // TM,)

    out = pl.pallas_call(
        _readout_kernel,
        grid=grid,
        in_specs=[
            pl.BlockSpec((TM, D), lambda i: (i, 0)),
            pl.BlockSpec((TM, T), lambda i: (i, 0)),
            pl.BlockSpec((T, D, E), lambda i: (0, 0, 0)),
            pl.BlockSpec((T, E), lambda i: (0, 0)),
        ],
        out_specs=pl.BlockSpec((T, TM, E), lambda i: (0, i, 0)),
        out_shape=jax.ShapeDtypeStruct((T, M, E), jnp.float32),
    )(X, onehot, W, b)
    return out.reshape(T, B, N, E)


# speedup vs baseline: 1.0731x; 1.0731x over previous
import jax
import jax.numpy as jnp
from jax.experimental import pallas as pl


def _readout_kernel(x_ref, oh_ref, w_ref, b_ref, out_ref):
    x = x_ref[0]
    oh = oh_ref[0]
    T = out_ref.shape[0]
    for t in range(T):
        acc = jnp.dot(x, w_ref[t], preferred_element_type=jnp.float32)
        mask = oh[:, t:t + 1]
        out_ref[t, 0] = (acc + b_ref[t]) * mask


def kernel(output_latents, output_task_index, W, b):
    B, N, D = output_latents.shape
    T, _, E = W.shape

    onehot = (output_task_index[..., None]
              == jnp.arange(T, dtype=output_task_index.dtype)).astype(
        jnp.float32)

    TN = 512
    grid = (B, N // TN)

    out = pl.pallas_call(
        _readout_kernel,
        grid=grid,
        in_specs=[
            pl.BlockSpec((1, TN, D), lambda b_, n: (b_, n, 0)),
            pl.BlockSpec((1, TN, T), lambda b_, n: (b_, n, 0)),
            pl.BlockSpec((T, D, E), lambda b_, n: (0, 0, 0)),
            pl.BlockSpec((T, E), lambda b_, n: (0, 0)),
        ],
        out_specs=pl.BlockSpec((T, 1, TN, E), lambda b_, n: (0, b_, n, 0)),
        out_shape=jax.ShapeDtypeStruct((T, B, N, E), jnp.float32),
    )(output_latents, onehot, W, b)
    return out


# speedup vs baseline: 1.1138x; 1.0380x over previous
import jax
import jax.numpy as jnp
from jax.experimental import pallas as pl
from jax.experimental.pallas import tpu as pltpu


def _readout_kernel(x_ref, oh_ref, w_ref, b_ref, out_ref):
    x = x_ref[0].astype(jnp.bfloat16)
    m = jnp.transpose(oh_ref[0], (1, 0))
    T = out_ref.shape[0]
    for t in range(T):
        acc = jnp.dot(x, w_ref[t], preferred_element_type=jnp.float32)
        out_ref[t, 0] = (acc + b_ref[t]) * m[:, t:t + 1]


def kernel(output_latents, output_task_index, W, b):
    B, N, D = output_latents.shape
    T, _, E = W.shape

    onehot = (output_task_index[:, None, :]
              == jnp.arange(T, dtype=output_task_index.dtype)[None, :, None]
              ).astype(jnp.float32)
    Wb = W.astype(jnp.bfloat16)

    TN = 512
    grid = (B, N // TN)

    out = pl.pallas_call(
        _readout_kernel,
        grid=grid,
        in_specs=[
            pl.BlockSpec((1, TN, D), lambda b_, n: (b_, n, 0)),
            pl.BlockSpec((1, T, TN), lambda b_, n: (b_, 0, n)),
            pl.BlockSpec((T, D, E), lambda b_, n: (0, 0, 0)),
            pl.BlockSpec((T, E), lambda b_, n: (0, 0)),
        ],
        out_specs=pl.BlockSpec((T, 1, TN, E), lambda b_, n: (0, b_, n, 0)),
        out_shape=jax.ShapeDtypeStruct((T, B, N, E), jnp.float32),
        compiler_params=pltpu.CompilerParams(
            dimension_semantics=("parallel", "parallel")),
    )(output_latents, onehot, Wb, b)
    return out
